# baseline (speedup 1.0000x reference)
import functools

import jax
import jax.numpy as jnp
import numpy as np
from jax import lax
from jax.experimental import pallas as pl
from jax.experimental.pallas import tpu as pltpu
from jax.experimental.pallas import tpu_sc as plsc

N = 10000
E = 160000
D = 16
DE = 16
HID = 32
G = 64
NCLS = 10

SC_CORES = 2
SC_SUBCORES = 16
SC_WORKERS = SC_CORES * SC_SUBCORES
CHUNK = 128
GRP = 8
E_PAD = 163840
NCH = E_PAD // CHUNK
CPW = NCH // SC_WORKERS
N_ACC = N + CHUNK
RPS = N // SC_SUBCORES

BLOCK_E = 2048
BLOCK_N = 2000

_SC_MESH = plsc.VectorSubcoreMesh(core_axis_name="c", subcore_axis_name="s")
_SC_PARAMS = pltpu.CompilerParams(use_tc_tiling_on_sc=False)


def _make_gather(d):
    @functools.partial(
        pl.kernel,
        out_type=jax.ShapeDtypeStruct((NCH, CHUNK, d), jnp.float32),
        mesh=_SC_MESH,
        compiler_params=_SC_PARAMS,
        scratch_types=[
            pltpu.VMEM((CPW, CHUNK), jnp.int32),
            pltpu.VMEM((GRP, CHUNK, d), jnp.float32),
            pltpu.SemaphoreType.DMA,
        ],
    )
    def gather_k(table_hbm, idx_hbm, out_hbm, idx_v, rows_v, sem):
        wid = lax.axis_index("s") * SC_CORES + lax.axis_index("c")
        base = wid * CPW
        pltpu.sync_copy(idx_hbm.at[pl.ds(base, CPW)], idx_v)

        def group(g, carry):
            c0 = base + g * GRP
            copies = [
                pltpu.async_copy(table_hbm.at[idx_v.at[g * GRP + b]], rows_v.at[b], sem)
                for b in range(GRP)
            ]
            for c in copies:
                c.wait()
            pltpu.sync_copy(rows_v, out_hbm.at[pl.ds(c0, GRP)])
            return carry

        lax.fori_loop(0, CPW // GRP, group, 0)

    return gather_k


def _make_scatter(d):
    @functools.partial(
        pl.kernel,
        out_type=jax.ShapeDtypeStruct((SC_CORES * N, d), jnp.float32),
        mesh=_SC_MESH,
        compiler_params=_SC_PARAMS,
        scratch_types=[
            pltpu.VMEM((CPW, CHUNK), jnp.int32),
            pltpu.VMEM((GRP, CHUNK, d), jnp.float32),
            pltpu.VMEM_SHARED((N_ACC, d), jnp.float32),
            pltpu.SemaphoreType.DMA,
        ],
    )
    def scatter_k(msg_hbm, dst_hbm, zeros_hbm, out_hbm, dst_v, rows_v, acc_sh, sem):
        cid = lax.axis_index("c")
        sid = lax.axis_index("s")
        wid = sid * SC_CORES + cid
        base = wid * CPW
        r0 = sid * RPS
        pltpu.sync_copy(zeros_hbm.at[pl.ds(r0, RPS)], acc_sh.at[pl.ds(r0, RPS)])
        pltpu.sync_copy(dst_hbm.at[pl.ds(base, CPW)], dst_v)
        plsc.subcore_barrier()

        def group(g, carry):
            c0 = base + g * GRP
            pltpu.sync_copy(msg_hbm.at[pl.ds(c0, GRP)], rows_v)
            for b in range(GRP):
                pltpu.sync_copy(rows_v.at[b], acc_sh.at[dst_v.at[g * GRP + b]], add=True)
            return carry

        lax.fori_loop(0, CPW // GRP, group, 0)
        plsc.subcore_barrier()
        pltpu.sync_copy(acc_sh.at[pl.ds(r0, RPS)], out_hbm.at[pl.ds(cid * N + r0, RPS)])

    return scatter_k


_gather16 = _make_gather(D)
_gather32 = _make_gather(HID)
_scatter32 = _make_scatter(HID)
_scatter16 = _make_scatter(D)


def _expand_mat(cin, cout):
    m = np.zeros((cin, cin * cout), np.float32)
    m[np.arange(cin).repeat(cout), np.arange(cin * cout)] = 1.0
    return jnp.asarray(m)


def _reduce_mat(cin, cout):
    m = np.zeros((cin * cout, cout), np.float32)
    m[np.arange(cin * cout), np.tile(np.arange(cout), cin)] = 1.0
    return jnp.asarray(m)


def _edge_dense(ea, xj, w1, b1, w2, b2, qm, rm, cout):
    e_tot = ea.shape[0]
    cin = xj.shape[1]
    kk = cin * cout
    nb = e_tot // BLOCK_E

    def body(ea_ref, xj_ref, w1_ref, b1_ref, w2_ref, b2_ref, qm_ref, rm_ref, out_ref):
        h = jnp.maximum(
            jnp.dot(ea_ref[...], w1_ref[...], preferred_element_type=jnp.float32)
            + b1_ref[...], 0.0)
        u = jnp.dot(h.astype(jnp.bfloat16), w2_ref[...].astype(jnp.bfloat16),
                    preferred_element_type=jnp.float32) + b2_ref[...]
        xr = jnp.dot(xj_ref[...].astype(jnp.bfloat16), qm_ref[...].astype(jnp.bfloat16),
                     preferred_element_type=jnp.float32)
        out_ref[...] = jnp.dot((xr * u).astype(jnp.bfloat16),
                               rm_ref[...].astype(jnp.bfloat16),
                               preferred_element_type=jnp.float32)

    return pl.pallas_call(
        body,
        grid=(nb,),
        in_specs=[
            pl.BlockSpec((BLOCK_E, DE), lambda i: (i, 0)),
            pl.BlockSpec((BLOCK_E, cin), lambda i: (i, 0)),
            pl.BlockSpec((DE, HID), lambda i: (0, 0)),
            pl.BlockSpec((1, HID), lambda i: (0, 0)),
            pl.BlockSpec((HID, kk), lambda i: (0, 0)),
            pl.BlockSpec((1, kk), lambda i: (0, 0)),
            pl.BlockSpec((cin, kk), lambda i: (0, 0)),
            pl.BlockSpec((kk, cout), lambda i: (0, 0)),
        ],
        out_specs=pl.BlockSpec((BLOCK_E, cout), lambda i: (i, 0)),
        out_shape=jax.ShapeDtypeStruct((e_tot, cout), jnp.float32),
    )(ea, xj, w1, b1, w2, b2, qm, rm)


def _combine(p, x, root, bias):
    cout = p.shape[2]
    cin = x.shape[1]
    nb = N // BLOCK_N

    def body(p_ref, x_ref, root_ref, bias_ref, out_ref):
        out_ref[...] = jnp.maximum(
            p_ref[0] + p_ref[1]
            + jnp.dot(x_ref[...], root_ref[...], preferred_element_type=jnp.float32)
            + bias_ref[...], 0.0)

    return pl.pallas_call(
        body,
        grid=(nb,),
        in_specs=[
            pl.BlockSpec((2, BLOCK_N, cout), lambda i: (0, i, 0)),
            pl.BlockSpec((BLOCK_N, cin), lambda i: (i, 0)),
            pl.BlockSpec((cin, cout), lambda i: (0, 0)),
            pl.BlockSpec((1, cout), lambda i: (0, 0)),
        ],
        out_specs=pl.BlockSpec((BLOCK_N, cout), lambda i: (i, 0)),
        out_shape=jax.ShapeDtypeStruct((N, cout), jnp.float32),
    )(p, x, root, bias)


def _final(p, h1, root, bias, batch2d, fcw, fcb, ow, ob):
    nb = N // BLOCK_N

    def body(p_ref, h1_ref, root_ref, bias_ref, b_ref, fcw_ref, fcb_ref,
             ow_ref, ob_ref, out_ref, acc):
        i = pl.program_id(0)
        h2 = jnp.maximum(
            p_ref[0] + p_ref[1]
            + jnp.dot(h1_ref[...], root_ref[...], preferred_element_type=jnp.float32)
            + bias_ref[...], 0.0)
        oh = (b_ref[0] == lax.broadcasted_iota(jnp.int32, (G, BLOCK_N), 0)
              ).astype(jnp.float32)
        part = jnp.dot(oh, h2, preferred_element_type=jnp.float32)

        @pl.when(i == 0)
        def _():
            acc[...] = part

        @pl.when(i > 0)
        def _():
            acc[...] = acc[...] + part

        @pl.when(i == nb - 1)
        def _():
            z = jnp.maximum(
                jnp.dot(acc[...], fcw_ref[...], preferred_element_type=jnp.float32)
                + fcb_ref[...], 0.0)
            out_ref[...] = jnp.dot(z, ow_ref[...], preferred_element_type=jnp.float32) + ob_ref[...]

    return pl.pallas_call(
        body,
        grid=(nb,),
        in_specs=[
            pl.BlockSpec((2, BLOCK_N, D), lambda i: (0, i, 0)),
            pl.BlockSpec((BLOCK_N, HID), lambda i: (i, 0)),
            pl.BlockSpec((HID, D), lambda i: (0, 0)),
            pl.BlockSpec((1, D), lambda i: (0, 0)),
            pl.BlockSpec((1, 1, BLOCK_N), lambda i: (i, 0, 0)),
            pl.BlockSpec((D, HID), lambda i: (0, 0)),
            pl.BlockSpec((1, HID), lambda i: (0, 0)),
            pl.BlockSpec((HID, NCLS), lambda i: (0, 0)),
            pl.BlockSpec((1, NCLS), lambda i: (0, 0)),
        ],
        out_specs=pl.BlockSpec((G, NCLS), lambda i: (0, 0)),
        out_shape=jax.ShapeDtypeStruct((G, NCLS), jnp.float32),
        scratch_shapes=[pltpu.VMEM((G, D), jnp.float32)],
    )(p, h1, root, bias, batch2d, fcw, fcb, ow, ob)


def kernel(x, edge_index, edge_attr, batch,
           c1_w1, c1_b1, c1_w2, c1_b2, c1_root, c1_bias,
           c2_w1, c2_b1, c2_w2, c2_b2, c2_root, c2_bias,
           fc_w, fc_b, out_w, out_b):
    pad = E_PAD - E
    src = jnp.pad(edge_index[0], (0, pad)).reshape(NCH, CHUNK)
    dst = jnp.pad(edge_index[1], (0, pad), constant_values=N).reshape(NCH, CHUNK)
    ea = jnp.pad(edge_attr, ((0, pad), (0, 0)))

    qm1 = _expand_mat(D, HID)
    rm1 = _reduce_mat(D, HID)
    qm2 = _expand_mat(HID, D)
    rm2 = _reduce_mat(HID, D)
    zeros32 = jnp.zeros((N, HID), jnp.float32)
    zeros16 = jnp.zeros((N, D), jnp.float32)

    xj1 = _gather16(x, src).reshape(E_PAD, D)
    msg1 = _edge_dense(ea, xj1, c1_w1, c1_b1.reshape(1, -1), c1_w2,
                       c1_b2.reshape(1, -1), qm1, rm1, HID)
    p1 = _scatter32(msg1.reshape(NCH, CHUNK, HID), dst, zeros32).reshape(2, N, HID)
    h1 = _combine(p1, x, c1_root, c1_bias.reshape(1, -1))

    xj2 = _gather32(h1, src).reshape(E_PAD, HID)
    msg2 = _edge_dense(ea, xj2, c2_w1, c2_b1.reshape(1, -1), c2_w2,
                       c2_b2.reshape(1, -1), qm2, rm2, D)
    p2 = _scatter16(msg2.reshape(NCH, CHUNK, D), dst, zeros16).reshape(2, N, D)

    return _final(p2, h1, c2_root, c2_bias.reshape(1, -1),
                  batch.reshape(N // BLOCK_N, 1, BLOCK_N),
                  fc_w, fc_b.reshape(1, -1), out_w, out_b.reshape(1, -1))


# speedup vs baseline: 1.0988x; 1.0988x over previous
import functools

import jax
import jax.numpy as jnp
import numpy as np
from jax import lax
from jax.experimental import pallas as pl
from jax.experimental.pallas import tpu as pltpu
from jax.experimental.pallas import tpu_sc as plsc

N = 10000
E = 160000
D = 16
DE = 16
HID = 32
G = 64
NCLS = 10

SC_CORES = 2
SC_SUBCORES = 16
SC_WORKERS = SC_CORES * SC_SUBCORES
CHUNK = 128
GRP = 8
E_PAD = 163840
NCH = E_PAD // CHUNK
CPW = NCH // SC_WORKERS
N_ACC = N + CHUNK
RPS = N // SC_SUBCORES

BLOCK_E = 4096
BLOCK_N = 2000

_SC_MESH = plsc.VectorSubcoreMesh(core_axis_name="c", subcore_axis_name="s")
_SC_PARAMS = pltpu.CompilerParams(use_tc_tiling_on_sc=False)


def _make_gather(d):
    @functools.partial(
        pl.kernel,
        out_type=jax.ShapeDtypeStruct((NCH, CHUNK, d), jnp.float32),
        mesh=_SC_MESH,
        compiler_params=_SC_PARAMS,
        scratch_types=[
            pltpu.VMEM((CPW, CHUNK), jnp.int32),
            pltpu.VMEM((GRP, CHUNK, d), jnp.float32),
            pltpu.SemaphoreType.DMA,
        ],
    )
    def gather_k(table_hbm, idx_hbm, out_hbm, idx_v, rows_v, sem):
        wid = lax.axis_index("s") * SC_CORES + lax.axis_index("c")
        base = wid * CPW
        pltpu.sync_copy(idx_hbm.at[pl.ds(base, CPW)], idx_v)

        def group(g, carry):
            c0 = base + g * GRP
            copies = [
                pltpu.async_copy(table_hbm.at[idx_v.at[g * GRP + b]], rows_v.at[b], sem)
                for b in range(GRP)
            ]
            for c in copies:
                c.wait()
            pltpu.sync_copy(rows_v, out_hbm.at[pl.ds(c0, GRP)])
            return carry

        lax.fori_loop(0, CPW // GRP, group, 0)

    return gather_k


def _make_scatter(d):
    @functools.partial(
        pl.kernel,
        out_type=jax.ShapeDtypeStruct((SC_CORES * N, d), jnp.float32),
        mesh=_SC_MESH,
        compiler_params=_SC_PARAMS,
        scratch_types=[
            pltpu.VMEM((CPW, CHUNK), jnp.int32),
            pltpu.VMEM((GRP, CHUNK, d), jnp.float32),
            pltpu.VMEM_SHARED((N_ACC, d), jnp.float32),
            pltpu.SemaphoreType.DMA,
        ],
    )
    def scatter_k(msg_hbm, dst_hbm, zeros_hbm, out_hbm, dst_v, rows_v, acc_sh, sem):
        cid = lax.axis_index("c")
        sid = lax.axis_index("s")
        wid = sid * SC_CORES + cid
        base = wid * CPW
        r0 = sid * RPS
        pltpu.sync_copy(zeros_hbm.at[pl.ds(r0, RPS)], acc_sh.at[pl.ds(r0, RPS)])
        pltpu.sync_copy(dst_hbm.at[pl.ds(base, CPW)], dst_v)
        plsc.subcore_barrier()

        def group(g, carry):
            c0 = base + g * GRP
            pltpu.sync_copy(msg_hbm.at[pl.ds(c0, GRP)], rows_v)
            for b in range(GRP):
                pltpu.sync_copy(rows_v.at[b], acc_sh.at[dst_v.at[g * GRP + b]], add=True)
            return carry

        lax.fori_loop(0, CPW // GRP, group, 0)
        plsc.subcore_barrier()
        pltpu.sync_copy(acc_sh.at[pl.ds(r0, RPS)], out_hbm.at[pl.ds(cid * N + r0, RPS)])

    return scatter_k


_gather16 = _make_gather(D)
_gather32 = _make_gather(HID)
_scatter32 = _make_scatter(HID)
_scatter16 = _make_scatter(D)


def _reduce_mat(cin, cout):
    m = np.zeros((cin * cout, cout), np.float32)
    m[np.arange(cin * cout), np.arange(cin * cout) // cin] = 1.0
    return jnp.asarray(m, dtype=jnp.bfloat16)


def _omajor_perm(cin, cout):
    c = np.arange(cin * cout)
    return jnp.asarray((c % cin) * cout + c // cin)


def _tile_mat(cin, cout):
    m = np.zeros((cin, cin * cout), np.float32)
    m[np.arange(cin * cout) % cin, np.arange(cin * cout)] = 1.0
    return jnp.asarray(m, dtype=jnp.bfloat16)


HAUG = 40


def _edge_dense(ea, xj, w1, b1, w2, b2, rm, cout):
    e_tot = ea.shape[0]
    cin = xj.shape[1]
    kk = cin * cout
    nb = e_tot // BLOCK_E
    perm = _omajor_perm(cin, cout)
    w1a = jnp.zeros((DE, HAUG), jnp.bfloat16).at[:, :HID].set(w1.astype(jnp.bfloat16))
    b1a = jnp.zeros((1, HAUG), jnp.float32).at[0, :HID].set(b1).at[0, HID].set(1.0)
    w2a = jnp.zeros((HAUG, kk), jnp.bfloat16).at[:HID].set(
        w2[:, perm].astype(jnp.bfloat16))
    w2a = w2a.at[HID].set(b2[perm].astype(jnp.bfloat16))

    use_repeat = cout <= 16
    tm = _tile_mat(cin, cout)

    def body(ea_ref, xj_ref, w1a_ref, b1a_ref, w2a_ref, tm_ref, rm_ref, out_ref):
        h = jnp.maximum(
            jnp.dot(ea_ref[...].astype(jnp.bfloat16), w1a_ref[...],
                    preferred_element_type=jnp.float32) + b1a_ref[...], 0.0)
        u = jnp.dot(h.astype(jnp.bfloat16), w2a_ref[...],
                    preferred_element_type=jnp.float32)
        if use_repeat:
            xt = pltpu.repeat(xj_ref[...], cout, axis=1)
        else:
            xt = jnp.dot(xj_ref[...].astype(jnp.bfloat16), tm_ref[...],
                         preferred_element_type=jnp.float32)
        out_ref[...] = jnp.dot((xt * u).astype(jnp.bfloat16), rm_ref[...],
                               preferred_element_type=jnp.float32)

    return pl.pallas_call(
        body,
        grid=(nb,),
        in_specs=[
            pl.BlockSpec((BLOCK_E, DE), lambda i: (i, 0)),
            pl.BlockSpec((BLOCK_E, cin), lambda i: (i, 0)),
            pl.BlockSpec((DE, HAUG), lambda i: (0, 0)),
            pl.BlockSpec((1, HAUG), lambda i: (0, 0)),
            pl.BlockSpec((HAUG, kk), lambda i: (0, 0)),
            pl.BlockSpec((cin, kk), lambda i: (0, 0)),
            pl.BlockSpec((kk, cout), lambda i: (0, 0)),
        ],
        out_specs=pl.BlockSpec((BLOCK_E, cout), lambda i: (i, 0)),
        out_shape=jax.ShapeDtypeStruct((e_tot, cout), jnp.float32),
    )(ea, xj, w1a, b1a, w2a, tm, rm)


def _combine(p, x, root, bias):
    cout = p.shape[2]
    cin = x.shape[1]
    nb = N // BLOCK_N

    def body(p_ref, x_ref, root_ref, bias_ref, out_ref):
        out_ref[...] = jnp.maximum(
            p_ref[0] + p_ref[1]
            + jnp.dot(x_ref[...], root_ref[...], preferred_element_type=jnp.float32)
            + bias_ref[...], 0.0)

    return pl.pallas_call(
        body,
        grid=(nb,),
        in_specs=[
            pl.BlockSpec((2, BLOCK_N, cout), lambda i: (0, i, 0)),
            pl.BlockSpec((BLOCK_N, cin), lambda i: (i, 0)),
            pl.BlockSpec((cin, cout), lambda i: (0, 0)),
            pl.BlockSpec((1, cout), lambda i: (0, 0)),
        ],
        out_specs=pl.BlockSpec((BLOCK_N, cout), lambda i: (i, 0)),
        out_shape=jax.ShapeDtypeStruct((N, cout), jnp.float32),
    )(p, x, root, bias)


def _final(p, h1, root, bias, batch2d, fcw, fcb, ow, ob):
    nb = N // BLOCK_N

    def body(p_ref, h1_ref, root_ref, bias_ref, b_ref, fcw_ref, fcb_ref,
             ow_ref, ob_ref, out_ref, acc):
        i = pl.program_id(0)
        h2 = jnp.maximum(
            p_ref[0] + p_ref[1]
            + jnp.dot(h1_ref[...], root_ref[...], preferred_element_type=jnp.float32)
            + bias_ref[...], 0.0)
        oh = (b_ref[0] == lax.broadcasted_iota(jnp.int32, (G, BLOCK_N), 0)
              ).astype(jnp.float32)
        part = jnp.dot(oh, h2, preferred_element_type=jnp.float32)

        @pl.when(i == 0)
        def _():
            acc[...] = part

        @pl.when(i > 0)
        def _():
            acc[...] = acc[...] + part

        @pl.when(i == nb - 1)
        def _():
            z = jnp.maximum(
                jnp.dot(acc[...], fcw_ref[...], preferred_element_type=jnp.float32)
                + fcb_ref[...], 0.0)
            out_ref[...] = jnp.dot(z, ow_ref[...], preferred_element_type=jnp.float32) + ob_ref[...]

    return pl.pallas_call(
        body,
        grid=(nb,),
        in_specs=[
            pl.BlockSpec((2, BLOCK_N, D), lambda i: (0, i, 0)),
            pl.BlockSpec((BLOCK_N, HID), lambda i: (i, 0)),
            pl.BlockSpec((HID, D), lambda i: (0, 0)),
            pl.BlockSpec((1, D), lambda i: (0, 0)),
            pl.BlockSpec((1, 1, BLOCK_N), lambda i: (i, 0, 0)),
            pl.BlockSpec((D, HID), lambda i: (0, 0)),
            pl.BlockSpec((1, HID), lambda i: (0, 0)),
            pl.BlockSpec((HID, NCLS), lambda i: (0, 0)),
            pl.BlockSpec((1, NCLS), lambda i: (0, 0)),
        ],
        out_specs=pl.BlockSpec((G, NCLS), lambda i: (0, 0)),
        out_shape=jax.ShapeDtypeStruct((G, NCLS), jnp.float32),
        scratch_shapes=[pltpu.VMEM((G, D), jnp.float32)],
    )(p, h1, root, bias, batch2d, fcw, fcb, ow, ob)


def kernel(x, edge_index, edge_attr, batch,
           c1_w1, c1_b1, c1_w2, c1_b2, c1_root, c1_bias,
           c2_w1, c2_b1, c2_w2, c2_b2, c2_root, c2_bias,
           fc_w, fc_b, out_w, out_b):
    pad = E_PAD - E
    src = jnp.pad(edge_index[0], (0, pad)).reshape(NCH, CHUNK)
    dst = jnp.pad(edge_index[1], (0, pad), constant_values=N).reshape(NCH, CHUNK)
    ea = jnp.pad(edge_attr, ((0, pad), (0, 0)))

    rm1 = _reduce_mat(D, HID)
    rm2 = _reduce_mat(HID, D)
    zeros32 = jnp.zeros((N, HID), jnp.float32)
    zeros16 = jnp.zeros((N, D), jnp.float32)

    xj1 = _gather16(x, src).reshape(E_PAD, D)
    msg1 = _edge_dense(ea, xj1, c1_w1, c1_b1, c1_w2, c1_b2, rm1, HID)
    p1 = _scatter32(msg1.reshape(NCH, CHUNK, HID), dst, zeros32).reshape(2, N, HID)
    h1 = _combine(p1, x, c1_root, c1_bias.reshape(1, -1))

    xj2 = _gather32(h1, src).reshape(E_PAD, HID)
    msg2 = _edge_dense(ea, xj2, c2_w1, c2_b1, c2_w2, c2_b2, rm2, D)
    p2 = _scatter16(msg2.reshape(NCH, CHUNK, D), dst, zeros16).reshape(2, N, D)

    return _final(p2, h1, c2_root, c2_bias.reshape(1, -1),
                  batch.reshape(N // BLOCK_N, 1, BLOCK_N),
                  fc_w, fc_b.reshape(1, -1), out_w, out_b.reshape(1, -1))
